# baseline (speedup 1.0000x reference)
import functools

import jax
import jax.numpy as jnp
from jax import lax
from jax.experimental import pallas as pl
from jax.experimental.pallas import tpu as pltpu
from jax.experimental.pallas import tpu_sc as plsc

NU = 25000
NN = 50000
NNP = 50048
HALF = 32
E = 800000
TILES = 16
SB = 28
CB = 14
PER_TILE = SB * CB * 128
E_PAD = TILES * PER_TILE
STRIPE = NNP // TILES
Q = 16384
QC = 8

_mesh = plsc.VectorSubcoreMesh(core_axis_name="c", subcore_axis_name="s")


@functools.partial(
    pl.kernel,
    out_type=(
        jax.ShapeDtypeStruct((2 * NNP, HALF), jnp.float32),
        jax.ShapeDtypeStruct((2 * NNP, HALF), jnp.float32),
        jax.ShapeDtypeStruct((2 * NNP, HALF), jnp.float32),
        jax.ShapeDtypeStruct((2 * Q,), jnp.float32),
    ),
    mesh=_mesh,
    compiler_params=pltpu.CompilerParams(use_tc_tiling_on_sc=False),
    scratch_types=[
        pltpu.VMEM((CB, 128), jnp.int32),
        pltpu.VMEM((CB, 128), jnp.int32),
        pltpu.VMEM((128, HALF), jnp.float32),
        pltpu.VMEM((128, HALF), jnp.float32),
        pltpu.VMEM((CB, 128), jnp.float32),
        pltpu.VMEM((QC, 128), jnp.int32),
        pltpu.VMEM((QC, 128), jnp.int32),
        pltpu.VMEM((128, HALF), jnp.float32),
        pltpu.VMEM((128, HALF), jnp.float32),
        pltpu.VMEM((128,), jnp.float32),
        pltpu.VMEM((32,), jnp.float32),
        pltpu.VMEM_SHARED((NNP, HALF), jnp.float32),
        pltpu.SemaphoreType.DMA,
        pltpu.SemaphoreType.DMA,
        pltpu.SemaphoreType.DMA,
        pltpu.SemaphoreType.DMA,
    ],
)
def _gcn(t0, srcb, dstb, valsb, uidx, iidx, zrows,
         t1, t2, t3, gout,
         src_v, dst_v, rows_a, rows_b, vals_v, uq_v, iq_v, ua_v, ia_v,
         gbuf, rbuf, acc,
         sem_ar, sem_br, sem_sa, sem_sb):
    c = lax.axis_index("c")
    s = lax.axis_index("s")
    tables = [t0, t1, t2, t3]

    def scale(rows, j):
        def scale_body(q, carry2):
            b = q * 32
            for h in range(2):
                vv = vals_v[j, pl.ds(b + h * 16, 16)]
                for t in range(16):
                    e = b + h * 16 + t
                    v = vv[t]
                    rows[e, pl.ds(0, 16)] = rows[e, pl.ds(0, 16)] * v
                    rows[e, pl.ds(16, 16)] = rows[e, pl.ds(16, 16)] * v
            return carry2

        lax.fori_loop(0, 4, scale_body, 0)

    for l in range(3):
        tin, tout = tables[l], tables[l + 1]
        pltpu.sync_copy(zrows, acc.at[pl.ds(s * STRIPE, STRIPE)])
        plsc.subcore_barrier()

        def sb_body(sb, carry0):
            pltpu.sync_copy(srcb.at[c, s, sb], src_v)
            pltpu.sync_copy(dstb.at[s, sb], dst_v)
            pltpu.sync_copy(valsb.at[s, sb], vals_v)

            def chunk_pair(jp, carry):
                ja = 2 * jp
                jb = 2 * jp + 1

                @pl.when(jp > 0)
                def _wait_sa():
                    pltpu.make_async_copy(rows_a, acc.at[dst_v.at[ja]],
                                          sem_sa).wait()

                pltpu.async_copy(tin.at[src_v.at[ja]], rows_a, sem_ar).wait()
                scale(rows_a, ja)
                pltpu.async_copy(rows_a, acc.at[dst_v.at[ja]], sem_sa,
                                 add=True)

                @pl.when(jp > 0)
                def _wait_sb():
                    pltpu.make_async_copy(rows_b, acc.at[dst_v.at[jb]],
                                          sem_sb).wait()

                pltpu.async_copy(tin.at[src_v.at[jb]], rows_b, sem_br).wait()
                scale(rows_b, jb)
                pltpu.async_copy(rows_b, acc.at[dst_v.at[jb]], sem_sb,
                                 add=True)
                return carry

            lax.fori_loop(0, CB // 2, chunk_pair, 0)
            pltpu.make_async_copy(rows_a, acc.at[dst_v.at[0]], sem_sa).wait()
            pltpu.make_async_copy(rows_b, acc.at[dst_v.at[0]], sem_sb).wait()
            return carry0

        lax.fori_loop(0, SB, sb_body, 0)
        plsc.subcore_barrier()
        pltpu.sync_copy(acc.at[pl.ds(s * STRIPE, STRIPE)],
                        tout.at[pl.ds(pl.multiple_of(c * NNP + s * STRIPE, 8),
                                      STRIPE)])
        plsc.subcore_barrier()

    pltpu.sync_copy(uidx.at[c, s], uq_v)
    pltpu.sync_copy(iidx.at[c, s], iq_v)
    rbuf[pl.ds(16, 16)] = lax.iota(jnp.int32, 16).astype(jnp.float32) * 0.0
    for ch in range(QC):
        pltpu.async_copy(tables[0].at[uq_v.at[ch]], ua_v, sem_ar).wait()
        pltpu.async_copy(tables[0].at[iq_v.at[ch]], ia_v, sem_ar).wait()
        for k in (1, 2, 3):
            pltpu.async_copy(tables[k].at[uq_v.at[ch]], rows_a, sem_ar).wait()
            pltpu.async_copy(tables[k].at[iq_v.at[ch]], rows_b, sem_ar).wait()

            def add_body(e, carry):
                ua_v[e, pl.ds(0, 16)] = (ua_v[e, pl.ds(0, 16)]
                                         + rows_a[e, pl.ds(0, 16)])
                ua_v[e, pl.ds(16, 16)] = (ua_v[e, pl.ds(16, 16)]
                                          + rows_a[e, pl.ds(16, 16)])
                ia_v[e, pl.ds(0, 16)] = (ia_v[e, pl.ds(0, 16)]
                                         + rows_b[e, pl.ds(0, 16)])
                ia_v[e, pl.ds(16, 16)] = (ia_v[e, pl.ds(16, 16)]
                                          + rows_b[e, pl.ds(16, 16)])
                return carry

            lax.fori_loop(0, 128, add_body, 0)

        def pair_body(g, carry):
            lanes = lax.iota(jnp.int32, 16)
            sv = lanes.astype(jnp.float32) * 0.0
            for t in range(16):
                e = g * 16 + t
                r = (ua_v[e, pl.ds(0, 16)] * ia_v[e, pl.ds(0, 16)]
                     + ua_v[e, pl.ds(16, 16)] * ia_v[e, pl.ds(16, 16)])
                for sh in (8, 4, 2, 1):
                    rbuf[pl.ds(0, 16)] = r
                    r = r + rbuf[pl.ds(sh, 16)]
                sv = jnp.where(lanes == t, r[0] * 0.0625, sv)
            gbuf[pl.ds(g * 16, 16)] = sv
            return carry

        lax.fori_loop(0, 8, pair_body, 0)
        pltpu.sync_copy(
            gbuf,
            gout.at[pl.ds(pl.multiple_of(c * Q + s * 1024 + ch * 128, 8), 128)])


def kernel(user_emb, item_emb, edge_index, edge_vals, users, items):
    src = edge_index[0].astype(jnp.int32)
    dst = edge_index[1].astype(jnp.int32)
    vals = edge_vals.astype(jnp.float32)
    npad = E_PAD - E
    pad_idx = (jnp.arange(npad, dtype=jnp.int32) * 997) % NN
    src_p = jnp.concatenate([src, pad_idx])
    dst_p = jnp.concatenate([dst, pad_idx])
    vals_p = jnp.concatenate([vals, jnp.zeros((npad,), jnp.float32)])
    src_r = src_p.reshape(TILES, SB, CB, 128)
    srcb = jnp.stack([src_r, src_r + NNP])
    dstb = dst_p.reshape(TILES, SB, CB, 128)
    valsb = vals_p.reshape(TILES, SB, CB, 128)
    u32 = users.astype(jnp.int32).reshape(TILES, QC, 128)
    i32 = items.astype(jnp.int32).reshape(TILES, QC, 128)
    uidx = jnp.stack([u32, u32 + NNP])
    iidx = jnp.stack([i32 + NU, i32 + NNP + NU])
    all0 = jnp.concatenate(
        [user_emb, item_emb, jnp.zeros((NNP - NN, HALF * 2), jnp.float32)],
        axis=0)
    t0 = jnp.concatenate([all0[:, :HALF], all0[:, HALF:]], axis=0)
    zrows = jnp.zeros((STRIPE, HALF), jnp.float32)
    t1, t2, t3, gout = _gcn(t0, srcb, dstb, valsb, uidx, iidx, zrows)
    return gout[:Q] + gout[Q:]


# speedup vs baseline: 1.4063x; 1.4063x over previous
import functools

import jax
import jax.numpy as jnp
from jax import lax
from jax.experimental import pallas as pl
from jax.experimental.pallas import tpu as pltpu
from jax.experimental.pallas import tpu_sc as plsc

NU = 25000
NN = 50000
NNP = 50048
HALF = 32
E = 800000
TILES = 16
SB = 28
CB = 14
PER_TILE = SB * CB * 128
E_PAD = TILES * PER_TILE
STRIPE = NNP // TILES
Q = 16384
QC = 8

_mesh = plsc.VectorSubcoreMesh(core_axis_name="c", subcore_axis_name="s")


@functools.partial(
    pl.kernel,
    out_type=(
        jax.ShapeDtypeStruct((2 * NNP, HALF), jnp.float32),
        jax.ShapeDtypeStruct((2 * NNP, HALF), jnp.float32),
        jax.ShapeDtypeStruct((2 * NNP, HALF), jnp.float32),
        jax.ShapeDtypeStruct((2 * Q,), jnp.float32),
    ),
    mesh=_mesh,
    compiler_params=pltpu.CompilerParams(use_tc_tiling_on_sc=False),
    scratch_types=[
        pltpu.VMEM((CB, 128), jnp.int32),
        pltpu.VMEM((CB, 128), jnp.int32),
        pltpu.VMEM((CB, 128), jnp.float32),
        pltpu.VMEM((128, HALF), jnp.float32),
        pltpu.VMEM((QC, 128), jnp.int32),
        pltpu.VMEM((QC, 128), jnp.int32),
        pltpu.VMEM((128, HALF), jnp.float32),
        pltpu.VMEM((128, HALF), jnp.float32),
        pltpu.VMEM((128, HALF), jnp.float32),
        pltpu.VMEM((128, HALF), jnp.float32),
        pltpu.VMEM((128,), jnp.float32),
        pltpu.VMEM((32,), jnp.float32),
        pltpu.VMEM_SHARED((NNP, HALF), jnp.float32),
        pltpu.SemaphoreType.DMA,
    ],
)
def _gcn(t0, srcb, dstb, valsb, uidx, iidx, zrows,
         t1, t2, t3, gout,
         src_v, dst_v, vals_v, rows_v, uq_v, iq_v, ua_v, ub_v, ia_v, ib_v,
         gbuf, rbuf, acc, sem):
    c = lax.axis_index("c")
    s = lax.axis_index("s")
    tables = [t0, t1, t2, t3]

    for l in range(3):
        tin, tout = tables[l], tables[l + 1]
        pltpu.sync_copy(zrows, acc.at[pl.ds(s * STRIPE, STRIPE)])
        plsc.subcore_barrier()

        def sb_body(sb, carry0):
            pltpu.sync_copy(srcb.at[c, s, sb], src_v)
            pltpu.sync_copy(dstb.at[s, sb], dst_v)
            pltpu.sync_copy(valsb.at[s, sb], vals_v)

            def chunk_body(j, carry):
                pltpu.async_copy(tin.at[src_v.at[j]], rows_v, sem).wait()

                def scale_body(eo, carry2):
                    vv = vals_v[j, pl.ds(eo * 16, 16)]
                    for t in range(16):
                        e = eo * 16 + t
                        v = vv[t]
                        rows_v[e, pl.ds(0, 16)] = rows_v[e, pl.ds(0, 16)] * v
                        rows_v[e, pl.ds(16, 16)] = rows_v[e, pl.ds(16, 16)] * v
                    return carry2

                lax.fori_loop(0, 8, scale_body, 0)
                pltpu.sync_copy(rows_v, acc.at[dst_v.at[j]], add=True)
                return carry

            lax.fori_loop(0, CB, chunk_body, 0)
            return carry0

        lax.fori_loop(0, SB, sb_body, 0)
        plsc.subcore_barrier()
        pltpu.sync_copy(acc.at[pl.ds(s * STRIPE, STRIPE)],
                        tout.at[pl.ds(pl.multiple_of(c * NNP + s * STRIPE, 8),
                                      STRIPE)])
        plsc.subcore_barrier()

    pltpu.sync_copy(uidx.at[c, s], uq_v)
    pltpu.sync_copy(iidx.at[c, s], iq_v)
    rbuf[pl.ds(16, 16)] = lax.iota(jnp.int32, 16).astype(jnp.float32) * 0.0
    for ch in range(QC):
        pltpu.async_copy(tables[0].at[uq_v.at[ch]], ua_v, sem).wait()
        pltpu.async_copy(tables[0].at[iq_v.at[ch]], ia_v, sem).wait()
        for k in (1, 2, 3):
            pltpu.async_copy(tables[k].at[uq_v.at[ch]], ub_v, sem).wait()
            pltpu.async_copy(tables[k].at[iq_v.at[ch]], ib_v, sem).wait()

            def add_body(e, carry):
                ua_v[e, pl.ds(0, 16)] = (ua_v[e, pl.ds(0, 16)]
                                         + ub_v[e, pl.ds(0, 16)])
                ua_v[e, pl.ds(16, 16)] = (ua_v[e, pl.ds(16, 16)]
                                          + ub_v[e, pl.ds(16, 16)])
                ia_v[e, pl.ds(0, 16)] = (ia_v[e, pl.ds(0, 16)]
                                         + ib_v[e, pl.ds(0, 16)])
                ia_v[e, pl.ds(16, 16)] = (ia_v[e, pl.ds(16, 16)]
                                          + ib_v[e, pl.ds(16, 16)])
                return carry

            lax.fori_loop(0, 128, add_body, 0)

        def pair_body(g, carry):
            lanes = lax.iota(jnp.int32, 16)
            sv = lanes.astype(jnp.float32) * 0.0
            for t in range(16):
                e = g * 16 + t
                r = (ua_v[e, pl.ds(0, 16)] * ia_v[e, pl.ds(0, 16)]
                     + ua_v[e, pl.ds(16, 16)] * ia_v[e, pl.ds(16, 16)])
                for sh in (8, 4, 2, 1):
                    rbuf[pl.ds(0, 16)] = r
                    r = r + rbuf[pl.ds(sh, 16)]
                sv = jnp.where(lanes == t, r[0] * 0.0625, sv)
            gbuf[pl.ds(g * 16, 16)] = sv
            return carry

        lax.fori_loop(0, 8, pair_body, 0)
        pltpu.sync_copy(
            gbuf,
            gout.at[pl.ds(pl.multiple_of(c * Q + s * 1024 + ch * 128, 8), 128)])


def kernel(user_emb, item_emb, edge_index, edge_vals, users, items):
    src = edge_index[0].astype(jnp.int32)
    dst = edge_index[1].astype(jnp.int32)
    vals = edge_vals.astype(jnp.float32)
    npad = E_PAD - E
    pad_idx = (jnp.arange(npad, dtype=jnp.int32) * 997) % NN
    src_p = jnp.concatenate([src, pad_idx])
    dst_p = jnp.concatenate([dst, pad_idx])
    vals_p = jnp.concatenate([vals, jnp.zeros((npad,), jnp.float32)])
    src_r = src_p.reshape(TILES, SB, CB, 128)
    srcb = jnp.stack([src_r, src_r + NNP])
    dstb = dst_p.reshape(TILES, SB, CB, 128)
    valsb = vals_p.reshape(TILES, SB, CB, 128)
    u32 = users.astype(jnp.int32).reshape(TILES, QC, 128)
    i32 = items.astype(jnp.int32).reshape(TILES, QC, 128)
    uidx = jnp.stack([u32, u32 + NNP])
    iidx = jnp.stack([i32 + NU, i32 + NNP + NU])
    all0 = jnp.concatenate(
        [user_emb, item_emb, jnp.zeros((NNP - NN, HALF * 2), jnp.float32)],
        axis=0)
    t0 = jnp.concatenate([all0[:, :HALF], all0[:, HALF:]], axis=0)
    zrows = jnp.zeros((STRIPE, HALF), jnp.float32)
    t1, t2, t3, gout = _gcn(t0, srcb, dstb, valsb, uidx, iidx, zrows)
    return gout[:Q] + gout[Q:]


# speedup vs baseline: 1.4456x; 1.0279x over previous
import functools

import jax
import jax.numpy as jnp
from jax import lax
from jax.experimental import pallas as pl
from jax.experimental.pallas import tpu as pltpu
from jax.experimental.pallas import tpu_sc as plsc

NU = 25000
NN = 50000
NNP = 50048
HALF = 32
E = 800000
TILES = 16
SB = 14
CB = 28
PER_TILE = SB * CB * 128
E_PAD = TILES * PER_TILE
STRIPE = NNP // TILES
Q = 16384
QC = 8

_mesh = plsc.VectorSubcoreMesh(core_axis_name="c", subcore_axis_name="s")


@functools.partial(
    pl.kernel,
    out_type=(
        jax.ShapeDtypeStruct((2 * NNP, HALF), jnp.float32),
        jax.ShapeDtypeStruct((2 * NNP, HALF), jnp.float32),
        jax.ShapeDtypeStruct((2 * NNP, HALF), jnp.float32),
        jax.ShapeDtypeStruct((2 * Q,), jnp.float32),
    ),
    mesh=_mesh,
    compiler_params=pltpu.CompilerParams(use_tc_tiling_on_sc=False),
    scratch_types=[
        pltpu.VMEM((CB, 128), jnp.int32),
        pltpu.VMEM((CB, 128), jnp.int32),
        pltpu.VMEM((CB, 128), jnp.float32),
        pltpu.VMEM((128, HALF), jnp.float32),
        pltpu.VMEM((QC, 128), jnp.int32),
        pltpu.VMEM((QC, 128), jnp.int32),
        pltpu.VMEM((128, HALF), jnp.float32),
        pltpu.VMEM((128, HALF), jnp.float32),
        pltpu.VMEM((128,), jnp.float32),
        pltpu.VMEM((32,), jnp.float32),
        pltpu.VMEM_SHARED((NNP, HALF), jnp.float32),
        pltpu.SemaphoreType.DMA,
    ],
)
def _gcn(t0, srcb, dstb, valsb, uidx, iidx, zrows,
         t1, t2, t3, gout,
         src_v, dst_v, vals_v, rows_v, uq_v, iq_v, ua_v, ia_v,
         gbuf, rbuf, acc, sem):
    c = lax.axis_index("c")
    s = lax.axis_index("s")
    tables = [t0, t1, t2, t3]

    for l in range(3):
        tin, tout = tables[l], tables[l + 1]
        pltpu.sync_copy(zrows, acc.at[pl.ds(s * STRIPE, STRIPE)])
        plsc.subcore_barrier()

        def sb_body(sb, carry0):
            pltpu.sync_copy(srcb.at[c, s, sb], src_v)
            pltpu.sync_copy(dstb.at[s, sb], dst_v)
            pltpu.sync_copy(valsb.at[s, sb], vals_v)

            def chunk_body(j, carry):
                pltpu.async_copy(tin.at[src_v.at[j]], rows_v, sem).wait()

                def scale_body(eo, carry2):
                    vv = vals_v[j, pl.ds(eo * 16, 16)]
                    for t in range(16):
                        e = eo * 16 + t
                        v = vv[t]
                        rows_v[e, pl.ds(0, 16)] = rows_v[e, pl.ds(0, 16)] * v
                        rows_v[e, pl.ds(16, 16)] = rows_v[e, pl.ds(16, 16)] * v
                    return carry2

                lax.fori_loop(0, 8, scale_body, 0)
                pltpu.sync_copy(rows_v, acc.at[dst_v.at[j]], add=True)
                return carry

            lax.fori_loop(0, CB, chunk_body, 0)
            return carry0

        lax.fori_loop(0, SB, sb_body, 0)
        plsc.subcore_barrier()
        pltpu.sync_copy(acc.at[pl.ds(s * STRIPE, STRIPE)],
                        tout.at[pl.ds(pl.multiple_of(c * NNP + s * STRIPE, 8),
                                      STRIPE)])
        plsc.subcore_barrier()

    pltpu.sync_copy(uidx.at[c, s], uq_v)
    pltpu.sync_copy(iidx.at[c, s], iq_v)
    rbuf[pl.ds(16, 16)] = lax.iota(jnp.int32, 16).astype(jnp.float32) * 0.0
    for ch in range(QC):
        pltpu.async_copy(tables[0].at[uq_v.at[ch]], ua_v, sem).wait()
        pltpu.async_copy(tables[0].at[iq_v.at[ch]], ia_v, sem).wait()
        for k in (1, 2, 3):
            pltpu.async_copy(tables[k].at[uq_v.at[ch]], rows_v, sem).wait()

            def add_body_u(e, carry):
                ua_v[e, pl.ds(0, 16)] = (ua_v[e, pl.ds(0, 16)]
                                         + rows_v[e, pl.ds(0, 16)])
                ua_v[e, pl.ds(16, 16)] = (ua_v[e, pl.ds(16, 16)]
                                          + rows_v[e, pl.ds(16, 16)])
                return carry

            lax.fori_loop(0, 128, add_body_u, 0)
            pltpu.async_copy(tables[k].at[iq_v.at[ch]], rows_v, sem).wait()

            def add_body_i(e, carry):
                ia_v[e, pl.ds(0, 16)] = (ia_v[e, pl.ds(0, 16)]
                                         + rows_v[e, pl.ds(0, 16)])
                ia_v[e, pl.ds(16, 16)] = (ia_v[e, pl.ds(16, 16)]
                                          + rows_v[e, pl.ds(16, 16)])
                return carry

            lax.fori_loop(0, 128, add_body_i, 0)

        def pair_body(g, carry):
            lanes = lax.iota(jnp.int32, 16)
            sv = lanes.astype(jnp.float32) * 0.0
            for t in range(16):
                e = g * 16 + t
                r = (ua_v[e, pl.ds(0, 16)] * ia_v[e, pl.ds(0, 16)]
                     + ua_v[e, pl.ds(16, 16)] * ia_v[e, pl.ds(16, 16)])
                for sh in (8, 4, 2, 1):
                    rbuf[pl.ds(0, 16)] = r
                    r = r + rbuf[pl.ds(sh, 16)]
                sv = jnp.where(lanes == t, r[0] * 0.0625, sv)
            gbuf[pl.ds(g * 16, 16)] = sv
            return carry

        lax.fori_loop(0, 8, pair_body, 0)
        pltpu.sync_copy(
            gbuf,
            gout.at[pl.ds(pl.multiple_of(c * Q + s * 1024 + ch * 128, 8), 128)])


def kernel(user_emb, item_emb, edge_index, edge_vals, users, items):
    src = edge_index[0].astype(jnp.int32)
    dst = edge_index[1].astype(jnp.int32)
    vals = edge_vals.astype(jnp.float32)
    npad = E_PAD - E
    pad_idx = (jnp.arange(npad, dtype=jnp.int32) * 997) % NN
    src_p = jnp.concatenate([src, pad_idx])
    dst_p = jnp.concatenate([dst, pad_idx])
    vals_p = jnp.concatenate([vals, jnp.zeros((npad,), jnp.float32)])
    src_r = src_p.reshape(TILES, SB, CB, 128)
    srcb = jnp.stack([src_r, src_r + NNP])
    dstb = dst_p.reshape(TILES, SB, CB, 128)
    valsb = vals_p.reshape(TILES, SB, CB, 128)
    u32 = users.astype(jnp.int32).reshape(TILES, QC, 128)
    i32 = items.astype(jnp.int32).reshape(TILES, QC, 128)
    uidx = jnp.stack([u32, u32 + NNP])
    iidx = jnp.stack([i32 + NU, i32 + NNP + NU])
    all0 = jnp.concatenate(
        [user_emb, item_emb, jnp.zeros((NNP - NN, HALF * 2), jnp.float32)],
        axis=0)
    t0 = jnp.concatenate([all0[:, :HALF], all0[:, HALF:]], axis=0)
    zrows = jnp.zeros((STRIPE, HALF), jnp.float32)
    t1, t2, t3, gout = _gcn(t0, srcb, dstb, valsb, uidx, iidx, zrows)
    return gout[:Q] + gout[Q:]
